# Initial kernel scaffold; baseline (speedup 1.0000x reference)
import functools

import jax
import jax.numpy as jnp
from jax import lax
from jax.experimental import pallas as pl
from jax.experimental.pallas import tpu as pltpu
from jax.experimental.pallas import tpu_sc as plsc

B = 16384
F = 26
NC = 2
NS = 16
NW = NC * NS
BPW = B // NW
CHUNK = BPW * F
PERIOD = 208
FIELD = 40000


def _sc_kernel(x_hbm, off_hbm, bias_hbm, tbl_hbm, out_hbm,
               idx_v, vals_v, off_v, bias_v, obuf_v, sem):
    wid = lax.axis_index("s") * NC + lax.axis_index("c")
    base = wid * CHUNK

    pltpu.sync_copy(x_hbm.at[pl.ds(base, CHUNK)], idx_v)
    pltpu.sync_copy(off_hbm, off_v)
    pltpu.sync_copy(bias_hbm, bias_v)

    def add_off(jj, _):
        for t in range(13):
            sl = pl.ds((jj * 13 + t) * 16, 16)
            idx_v[sl] = idx_v[sl] + off_v[pl.ds(t * 16, 16)]
        return _
    lax.fori_loop(0, CHUNK // PERIOD, add_off, 0)

    pltpu.async_copy(tbl_hbm.at[idx_v], vals_v, sem).wait()

    iota = lax.iota(jnp.int32, 16)
    bias16 = bias_v[...]

    def reduce16(c, _):
        p = (c * (16 * F)) + iota * F
        acc = plsc.load_gather(vals_v, [p])
        for f in range(1, F):
            acc = acc + plsc.load_gather(vals_v, [p + f])
        obuf_v[pl.ds(c * 16, 16)] = acc + bias16
        return _
    lax.fori_loop(0, BPW // 16, reduce16, 0)

    pltpu.sync_copy(obuf_v, out_hbm.at[pl.ds(wid * BPW, BPW)])


@jax.jit
def kernel(x, table, bias):
    x_flat = x.reshape(-1).astype(jnp.int32)
    tbl_flat = table.reshape(-1)
    off208 = (jnp.arange(PERIOD, dtype=jnp.int32) % F) * FIELD
    bias16 = jnp.broadcast_to(bias.astype(jnp.float32), (16,))

    run = functools.partial(
        pl.kernel,
        mesh=plsc.VectorSubcoreMesh(core_axis_name="c", subcore_axis_name="s"),
        out_type=jax.ShapeDtypeStruct((B,), jnp.float32),
        scratch_types=[
            pltpu.VMEM((CHUNK,), jnp.int32),
            pltpu.VMEM((CHUNK,), jnp.float32),
            pltpu.VMEM((PERIOD,), jnp.int32),
            pltpu.VMEM((16,), jnp.float32),
            pltpu.VMEM((BPW,), jnp.float32),
            pltpu.SemaphoreType.DMA,
        ],
    )(_sc_kernel)

    out = run(x_flat, off208, bias16, tbl_flat)
    return out.reshape(B, 1)


# speedup vs baseline: 1.4814x; 1.4814x over previous
import functools

import jax
import jax.numpy as jnp
from jax import lax
from jax.experimental import pallas as pl
from jax.experimental.pallas import tpu as pltpu
from jax.experimental.pallas import tpu_sc as plsc

B = 16384
F = 26
NC = 2
NS = 16
NW = NC * NS
BPW = B // NW
CHUNK = BPW * F
FIELD = 40000


def _sc_kernel(xt_hbm, bias_hbm, tbl_hbm, out_hbm,
               idx_v, vals_v, bias_v, obuf_v, sem):
    wid = lax.axis_index("c") * NS + lax.axis_index("s")

    descs = [
        pltpu.async_copy(
            xt_hbm.at[pl.ds(f * B + wid * BPW, BPW)],
            idx_v.at[pl.ds(f * BPW, BPW)],
            sem,
        )
        for f in range(F)
    ]
    pltpu.sync_copy(bias_hbm, bias_v)
    for d in descs:
        d.wait()

    def add_off(i, _):
        for f in range(1, F):
            sl = pl.ds(f * BPW + i * 16, 16)
            idx_v[sl] = idx_v[sl] + (f * FIELD)
        return _
    lax.fori_loop(0, BPW // 16, add_off, 0)

    pltpu.async_copy(tbl_hbm.at[idx_v], vals_v, sem).wait()

    bias16 = bias_v[...]

    def reduce16(c, _):
        r = c * 16
        acc = vals_v[pl.ds(r, 16)]
        for f in range(1, F):
            acc = acc + vals_v[pl.ds(f * BPW + r, 16)]
        obuf_v[pl.ds(r, 16)] = acc + bias16
        return _
    lax.fori_loop(0, BPW // 16, reduce16, 0)

    pltpu.sync_copy(obuf_v, out_hbm.at[pl.ds(wid * BPW, BPW)])


@jax.jit
def kernel(x, table, bias):
    xt_flat = x.astype(jnp.int32).T.reshape(-1)
    tbl_flat = table.reshape(-1)
    bias16 = jnp.broadcast_to(bias.astype(jnp.float32), (16,))

    run = functools.partial(
        pl.kernel,
        mesh=plsc.VectorSubcoreMesh(core_axis_name="c", subcore_axis_name="s"),
        out_type=jax.ShapeDtypeStruct((B,), jnp.float32),
        scratch_types=[
            pltpu.VMEM((CHUNK,), jnp.int32),
            pltpu.VMEM((CHUNK,), jnp.float32),
            pltpu.VMEM((16,), jnp.float32),
            pltpu.VMEM((BPW,), jnp.float32),
            pltpu.SemaphoreType.DMA,
        ],
    )(_sc_kernel)

    out = run(xt_flat, bias16, tbl_flat)
    return out.reshape(B, 1)
